# baseline (speedup 1.0000x reference)
import functools

import numpy as np
import jax
import jax.numpy as jnp
from jax import lax
from jax.experimental import pallas as pl
from jax.experimental.pallas import tpu as pltpu
from jax.experimental.pallas import tpu_sc as plsc

ANG = 0.52917721092
FSC = 0.0072973525693
N_NODES = 100000
N_EDGES = 6400000
N_SPECIES = 87

NW = 32
CHUNK_E = 2048
N_CHUNKS = N_EDGES // CHUNK_E
BASE_CHUNKS = N_CHUNKS // NW
EXTRA_CHUNKS = N_CHUNKS % NW
NPAD = 100096
SEG = NPAD // 16
NSW = N_NODES // 4


def _pair_tables():
    c6 = np.linspace(1.5, 500.0, N_SPECIES).astype(np.float32).astype(np.float64)
    al = np.linspace(1.0, 60.0, N_SPECIES).astype(np.float32).astype(np.float64)
    c6i, c6j = c6[:, None], c6[None, :]
    ai, aj = al[:, None], al[None, :]
    aij = 0.5 * (ai + aj)
    c6ij = 2 * ai * aj * c6i * c6j / (c6i * aj**2 + c6j * ai**2)
    Re = (aij * (128.0 / FSC ** (4.0 / 3.0))) ** (1.0 / 7.0)
    Re2 = Re * Re
    Re4 = Re2 * Re2
    muw = (0.483053463 - 0.0376191669 * Re + 0.00127066988 * Re2
           - 7.21940151e-07 * Re4) / (
        0.038421212 - 0.0316915319 * Re + 0.023741089 * Re2)
    c8ij = 5 * c6ij / muw
    c10ij = 245 * c6ij / (8 * muw**2)
    w = 4 * c6ij / (3 * aij**2)
    q2 = aij * muw * w
    ze = 0.5 * muw * Re2
    eze = np.exp(-ze)
    s6 = eze * (1.0 + ze + 0.5 * ze**2 + ze**3 / 6.0)
    f6e = 1.0 - s6
    muwRe = muw * Re
    df6e = muwRe * s6 - eze * (muwRe + 0.5 * Re * muwRe**2
                               + 1.0 / 8.0 * Re2 * muwRe**3)
    s8 = 1.0 / 24.0 * eze * ze**4
    f8e = f6e - s8
    df8e = df6e + muwRe * s8 - 1.0 / 48.0 * eze * Re2 * Re * muwRe**4
    s10 = 1.0 / 120.0 * eze * ze**5
    f10e = f8e - s10
    df10e = df8e + muwRe * s10 - 1.0 / 384.0 * eze * Re2 * Re2 * muwRe**5
    den = 2 * c6ij * Re2 * (6 * f6e - Re * df6e)
    A = (0.5 + c8ij * (8 * f8e - Re * df8e) / den
         + c10ij * (10 * f10e - Re * df10e) / (den * Re2))
    aq2 = A * q2
    MU = -0.5 * muw / ANG**2
    C6 = -0.5 * c6ij * ANG**6
    C8 = -0.5 * c8ij * ANG**8
    C10 = -0.5 * c10ij * ANG**10
    AQ = 0.5 * aq2 * ANG
    f32 = lambda x: np.asarray(x, np.float32)
    return (f32(MU).ravel(), f32(C6).ravel(), f32(C8).ravel(),
            f32(C10).ravel(), f32(AQ).ravel())


_TBL_MUW, _TBL_C6, _TBL_C8, _TBL_C10, _TBL_AQ2 = _pair_tables()


def _sc_body(specw_h, src_h, dst_h, dist_h, sw_h,
             tmu_h, tc6_h, tc8_h, tc10_h, taq_h,
             edisp_o, ex_o,
             specw_v, tmu_v, tc6_v, tc8_v, tc10_v, taq_v,
             srcidx_0, dstidx_0, dist_0, sw_0, e1_0, e2_0,
             srcidx_1, dstidx_1, dist_1, sw_1, e1_1, e2_1,
             srcidx_2, dstidx_2, dist_2, sw_2, e1_2, e2_2,
             srcidx_3, dstidx_3, dist_3, sw_3, e1_3, e2_3,
             acc1, acc2,
             semL0, semL1, semL2, semL3, semS0, semS1, semS2, semS3):
    srcidx = (srcidx_0, srcidx_1, srcidx_2, srcidx_3)
    dstidx = (dstidx_0, dstidx_1, dstidx_2, dstidx_3)
    dist = (dist_0, dist_1, dist_2, dist_3)
    sw = (sw_0, sw_1, sw_2, sw_3)
    e1 = (e1_0, e1_1, e1_2, e1_3)
    e2 = (e2_0, e2_1, e2_2, e2_3)
    semL = (semL0, semL1, semL2, semL3)
    semS = (semS0, semS1, semS2, semS3)
    cid = lax.axis_index("c")
    sid = lax.axis_index("s")
    wid = sid * 2 + cid

    pltpu.sync_copy(specw_h, specw_v)
    pltpu.sync_copy(tmu_h, tmu_v)
    pltpu.sync_copy(tc6_h, tc6_v)
    pltpu.sync_copy(tc8_h, tc8_v)
    pltpu.sync_copy(tc10_h, tc10_v)
    pltpu.sync_copy(taq_h, taq_v)

    def _zb(i, carry):
        e1_0[pl.ds(i * 16, 16)] = jnp.zeros((16,), jnp.float32)
        return carry
    lax.fori_loop(0, CHUNK_E // 16, _zb, 0)
    for acc in (acc1, acc2):
        for k in range(3):
            pltpu.sync_copy(e1_0, acc.at[pl.ds(sid * SEG + k * CHUNK_E,
                                               CHUNK_E)])
        pltpu.sync_copy(e1_0.at[pl.ds(0, SEG - 3 * CHUNK_E)],
                        acc.at[pl.ds(sid * SEG + 3 * CHUNK_E,
                                     SEG - 3 * CHUNK_E)])
    plsc.subcore_barrier()

    nchunks = BASE_CHUNKS + jnp.where(wid < EXTRA_CHUNKS, 1, 0)

    def _fire_lin(b, c):
        sl_e = pl.ds((wid + NW * c) * CHUNK_E, CHUNK_E)
        pltpu.async_copy(src_h.at[sl_e], srcidx[b], semL[b])
        pltpu.async_copy(dst_h.at[sl_e], dstidx[b], semL[b])
        pltpu.async_copy(dist_h.at[sl_e], dist[b], semL[b])
        pltpu.async_copy(sw_h.at[sl_e], sw[b], semL[b])

    def _wait_lin(b):
        sl_e = pl.ds(0, CHUNK_E)
        pltpu.make_async_copy(src_h.at[sl_e], srcidx[b], semL[b]).wait()
        pltpu.make_async_copy(dst_h.at[sl_e], dstidx[b], semL[b]).wait()
        pltpu.make_async_copy(dist_h.at[sl_e], dist[b], semL[b]).wait()
        pltpu.make_async_copy(sw_h.at[sl_e], sw[b], semL[b]).wait()

    def _fire_scat(b):
        pltpu.async_copy(e1[b], acc1.at[srcidx[b]], semS[b], add=True)
        pltpu.async_copy(e2[b], acc2.at[srcidx[b]], semS[b], add=True)

    def _wait_scat(b):
        pltpu.make_async_copy(e1[b], acc1.at[srcidx[b]], semS[b]).wait()
        pltpu.make_async_copy(e2[b], acc2.at[srcidx[b]], semS[b]).wait()

    def _species(idx_vec):
        word = plsc.load_gather(specw_v, [jnp.right_shift(idx_vec, 2)])
        sh = jnp.left_shift(jnp.bitwise_and(idx_vec, 3), 3)
        return jnp.bitwise_and(jnp.right_shift(word, sh), 255)

    def _compute(b):
        srcidx_b, dstidx_b, dist_b, sw_b, e1_b, e2_b = (
            srcidx[b], dstidx[b], dist[b], sw[b], e1[b], e2[b])

        @plsc.parallel_loop(0, CHUNK_E // 16, unroll=8)
        def _vec(v):
            sl = pl.ds(v * 16, 16)
            si = _species(srcidx_b[sl])
            sj = _species(dstidx_b[sl])
            p = si * N_SPECIES + sj
            d = dist_b[sl]
            swv = sw_b[sl]
            mu = plsc.load_gather(tmu_v, [p])
            c6 = plsc.load_gather(tc6_v, [p])
            c8 = plsc.load_gather(tc8_v, [p])
            c10 = plsc.load_gather(tc10_v, [p])
            aq2 = plsc.load_gather(taq_v, [p])
            zn = mu * (d * d)
            ez = jnp.exp(zn)
            a1 = ez * zn
            a2 = a1 * zn
            a3 = a2 * zn
            a4 = a3 * zn
            a5 = a4 * zn
            f6 = 1.0 - ez + a1 - 0.5 * a2 + (1.0 / 6.0) * a3
            f8 = f6 - (1.0 / 24.0) * a4
            f10 = f8 + (1.0 / 120.0) * a5
            inv = 1.0 / d
            inv2 = inv * inv
            inv6 = inv2 * inv2 * inv2
            e1_b[sl] = swv * inv6 * (f6 * c6 + inv2 * (f8 * c8
                                                      + inv2 * (f10 * c10)))
            e2_b[sl] = aq2 * ez * inv * swv

    _fire_lin(0, 0)
    _fire_lin(1, 1)
    _fire_lin(2, 2)

    def _body(b, c):
        @pl.when(c < nchunks)
        def _():
            _wait_lin(b)
            _compute(b)
            _fire_scat(b)

            @pl.when(c >= 1)
            def _():
                _wait_scat((b + 3) % 4)

            @pl.when(c + 3 < nchunks)
            def _():
                _fire_lin((b + 3) % 4, c + 3)

    def _quad(i, carry):
        _body(0, 4 * i)
        _body(1, 4 * i + 1)
        _body(2, 4 * i + 2)
        _body(3, 4 * i + 3)
        return carry
    lax.fori_loop(0, (BASE_CHUNKS + 4) // 4, _quad, 0)

    for k in range(4):
        @pl.when((nchunks - 1) % 4 == k)
        def _(k=k):
            _wait_scat(k)

    plsc.subcore_barrier()
    for acc, out in ((acc1, edisp_o), (acc2, ex_o)):
        for k in range(4):
            n = CHUNK_E if k < 3 else SEG - 3 * CHUNK_E
            src_sl = pl.ds(sid * SEG + k * CHUNK_E, n)
            dst_sl = pl.ds(cid * NPAD + sid * SEG + k * CHUNK_E, n)
            bb = e1_0 if n == CHUNK_E else e1_0.at[pl.ds(0, n)]
            pltpu.sync_copy(acc.at[src_sl], bb)
            pltpu.sync_copy(bb, out.at[dst_sl])


def _combine_body(e1_ref, e2_ref, edisp_ref, ex_ref, tot_ref):
    a = e1_ref[0] + e1_ref[1]
    b = e2_ref[0] + e2_ref[1]
    edisp_ref[...] = a
    ex_ref[...] = b
    tot_ref[...] = a + b


def kernel(species, edge_src, edge_dst, distances, switch):
    src1 = edge_src.astype(jnp.int32)
    dst1 = edge_dst.astype(jnp.int32)
    s4 = species.astype(jnp.int32).reshape(NSW, 4)
    specw = (s4[:, 0] | (s4[:, 1] << 8) | (s4[:, 2] << 16) | (s4[:, 3] << 24))

    mesh = plsc.VectorSubcoreMesh(core_axis_name="c", subcore_axis_name="s")
    f32 = jnp.float32
    sc = functools.partial(
        pl.kernel, mesh=mesh,
        compiler_params=pltpu.CompilerParams(needs_layout_passes=False),
        out_type=(jax.ShapeDtypeStruct((2 * NPAD,), f32),
                  jax.ShapeDtypeStruct((2 * NPAD,), f32)),
        scratch_types=[
            pltpu.VMEM((NSW,), jnp.int32),
            pltpu.VMEM((N_SPECIES * N_SPECIES,), f32),
            pltpu.VMEM((N_SPECIES * N_SPECIES,), f32),
            pltpu.VMEM((N_SPECIES * N_SPECIES,), f32),
            pltpu.VMEM((N_SPECIES * N_SPECIES,), f32),
            pltpu.VMEM((N_SPECIES * N_SPECIES,), f32),
        ] + [
            pltpu.VMEM((CHUNK_E,), jnp.int32),
            pltpu.VMEM((CHUNK_E,), jnp.int32),
            pltpu.VMEM((CHUNK_E,), f32),
            pltpu.VMEM((CHUNK_E,), f32),
            pltpu.VMEM((CHUNK_E,), f32),
            pltpu.VMEM((CHUNK_E,), f32),
        ] * 4 + [
            pltpu.VMEM_SHARED((NPAD,), f32),
            pltpu.VMEM_SHARED((NPAD,), f32),
            pltpu.SemaphoreType.DMA,
            pltpu.SemaphoreType.DMA,
            pltpu.SemaphoreType.DMA,
            pltpu.SemaphoreType.DMA,
            pltpu.SemaphoreType.DMA,
            pltpu.SemaphoreType.DMA,
            pltpu.SemaphoreType.DMA,
            pltpu.SemaphoreType.DMA,
        ])(_sc_body)

    edisp_p, ex_p = sc(
        specw, src1, dst1, distances, switch,
        jnp.asarray(_TBL_MUW), jnp.asarray(_TBL_C6), jnp.asarray(_TBL_C8),
        jnp.asarray(_TBL_C10), jnp.asarray(_TBL_AQ2))

    e1_3d = edisp_p.reshape(2, NPAD // 128, 128)
    e2_3d = ex_p.reshape(2, NPAD // 128, 128)
    edisp_pad, ex_pad, tot_pad = pl.pallas_call(
        _combine_body,
        out_shape=(jax.ShapeDtypeStruct((NPAD // 128, 128), f32),) * 3,
    )(e1_3d, e2_3d)
    edisp = edisp_pad.reshape(NPAD)[:N_NODES]
    ex = ex_pad.reshape(NPAD)[:N_NODES]
    tot = tot_pad.reshape(NPAD)[:N_NODES]
    return (edisp, ex, tot)


# speedup vs baseline: 1.0214x; 1.0214x over previous
import functools

import numpy as np
import jax
import jax.numpy as jnp
from jax import lax
from jax.experimental import pallas as pl
from jax.experimental.pallas import tpu as pltpu
from jax.experimental.pallas import tpu_sc as plsc

ANG = 0.52917721092
FSC = 0.0072973525693
N_NODES = 100000
N_EDGES = 6400000
N_SPECIES = 87

NW = 32
CHUNK_E = 2048
N_CHUNKS = N_EDGES // CHUNK_E
BASE_CHUNKS = N_CHUNKS // NW
EXTRA_CHUNKS = N_CHUNKS % NW
NPAD = 100096
SEG = NPAD // 16
NSW = N_NODES // 4
N_ROWS = N_EDGES // 128
RPC = CHUNK_E // 128


def _pair_tables():
    c6 = np.linspace(1.5, 500.0, N_SPECIES).astype(np.float32).astype(np.float64)
    al = np.linspace(1.0, 60.0, N_SPECIES).astype(np.float32).astype(np.float64)
    c6i, c6j = c6[:, None], c6[None, :]
    ai, aj = al[:, None], al[None, :]
    aij = 0.5 * (ai + aj)
    c6ij = 2 * ai * aj * c6i * c6j / (c6i * aj**2 + c6j * ai**2)
    Re = (aij * (128.0 / FSC ** (4.0 / 3.0))) ** (1.0 / 7.0)
    Re2 = Re * Re
    Re4 = Re2 * Re2
    muw = (0.483053463 - 0.0376191669 * Re + 0.00127066988 * Re2
           - 7.21940151e-07 * Re4) / (
        0.038421212 - 0.0316915319 * Re + 0.023741089 * Re2)
    c8ij = 5 * c6ij / muw
    c10ij = 245 * c6ij / (8 * muw**2)
    w = 4 * c6ij / (3 * aij**2)
    q2 = aij * muw * w
    ze = 0.5 * muw * Re2
    eze = np.exp(-ze)
    s6 = eze * (1.0 + ze + 0.5 * ze**2 + ze**3 / 6.0)
    f6e = 1.0 - s6
    muwRe = muw * Re
    df6e = muwRe * s6 - eze * (muwRe + 0.5 * Re * muwRe**2
                               + 1.0 / 8.0 * Re2 * muwRe**3)
    s8 = 1.0 / 24.0 * eze * ze**4
    f8e = f6e - s8
    df8e = df6e + muwRe * s8 - 1.0 / 48.0 * eze * Re2 * Re * muwRe**4
    s10 = 1.0 / 120.0 * eze * ze**5
    f10e = f8e - s10
    df10e = df8e + muwRe * s10 - 1.0 / 384.0 * eze * Re2 * Re2 * muwRe**5
    den = 2 * c6ij * Re2 * (6 * f6e - Re * df6e)
    A = (0.5 + c8ij * (8 * f8e - Re * df8e) / den
         + c10ij * (10 * f10e - Re * df10e) / (den * Re2))
    aq2 = A * q2
    MU = -0.5 * muw / ANG**2
    C6 = -0.5 * c6ij * ANG**6
    C8 = -0.5 * c8ij * ANG**8
    C10 = -0.5 * c10ij * ANG**10
    AQ = 0.5 * aq2 * ANG
    f32 = lambda x: np.asarray(x, np.float32)
    return (f32(MU).ravel(), f32(C6).ravel(), f32(C8).ravel(),
            f32(C10).ravel(), f32(AQ).ravel())


_TBL_MUW, _TBL_C6, _TBL_C8, _TBL_C10, _TBL_AQ2 = _pair_tables()


def _sc_body(specw_h, src_h, dst_h, dist_h, sw_h,
             tmu_h, tc6_h, tc8_h, tc10_h, taq_h,
             edisp_o, ex_o,
             specw_v, tmu_v, tc6_v, tc8_v, tc10_v, taq_v,
             srcidx_0, dstidx_0, dist_0, sw_0, scidx_0, e1_0, e2_0,
             srcidx_1, dstidx_1, dist_1, sw_1, scidx_1, e1_1, e2_1,
             srcidx_2, dstidx_2, dist_2, sw_2, scidx_2, e1_2, e2_2,
             zbuf, acc1, acc2,
             semL0, semL1, semL2, semS0, semS1, semS2):
    srcidx = (srcidx_0, srcidx_1, srcidx_2)
    dstidx = (dstidx_0, dstidx_1, dstidx_2)
    dist = (dist_0, dist_1, dist_2)
    sw = (sw_0, sw_1, sw_2)
    scidx = (scidx_0, scidx_1, scidx_2)
    e1 = (e1_0, e1_1, e1_2)
    e2 = (e2_0, e2_1, e2_2)
    semL = (semL0, semL1, semL2)
    semS = (semS0, semS1, semS2)
    cid = lax.axis_index("c")
    sid = lax.axis_index("s")
    wid = sid * 2 + cid

    pltpu.sync_copy(specw_h, specw_v)
    pltpu.sync_copy(tmu_h, tmu_v)
    pltpu.sync_copy(tc6_h, tc6_v)
    pltpu.sync_copy(tc8_h, tc8_v)
    pltpu.sync_copy(tc10_h, tc10_v)
    pltpu.sync_copy(taq_h, taq_v)

    def _zb(i, carry):
        zbuf[pl.ds(i * 16, 16)] = jnp.zeros((16,), jnp.float32)
        return carry
    lax.fori_loop(0, SEG // 16, _zb, 0)
    pltpu.sync_copy(zbuf, acc1.at[pl.ds(sid * SEG, SEG)])
    pltpu.sync_copy(zbuf, acc2.at[pl.ds(sid * SEG, SEG)])
    plsc.subcore_barrier()

    nchunks = BASE_CHUNKS + jnp.where(wid < EXTRA_CHUNKS, 1, 0)

    def _fire_lin(b, c):
        sl_e = pl.ds((wid + NW * c) * RPC, RPC)
        pltpu.async_copy(src_h.at[sl_e], srcidx[b], semL[b])
        pltpu.async_copy(dst_h.at[sl_e], dstidx[b], semL[b])
        pltpu.async_copy(dist_h.at[sl_e], dist[b], semL[b])
        pltpu.async_copy(sw_h.at[sl_e], sw[b], semL[b])

    def _wait_lin(b):
        sl_e = pl.ds(0, RPC)
        pltpu.make_async_copy(src_h.at[sl_e], srcidx[b], semL[b]).wait()
        pltpu.make_async_copy(dst_h.at[sl_e], dstidx[b], semL[b]).wait()
        pltpu.make_async_copy(dist_h.at[sl_e], dist[b], semL[b]).wait()
        pltpu.make_async_copy(sw_h.at[sl_e], sw[b], semL[b]).wait()

    def _fire_scat(b):
        pltpu.async_copy(e1[b], acc1.at[scidx[b]], semS[b], add=True)
        pltpu.async_copy(e2[b], acc2.at[scidx[b]], semS[b], add=True)

    def _wait_scat(b):
        pltpu.make_async_copy(e1[b], acc1.at[scidx[b]], semS[b]).wait()
        pltpu.make_async_copy(e2[b], acc2.at[scidx[b]], semS[b]).wait()

    def _species(idx_vec):
        word = plsc.load_gather(specw_v, [jnp.right_shift(idx_vec, 2)])
        sh = jnp.left_shift(jnp.bitwise_and(idx_vec, 3), 3)
        return jnp.bitwise_and(jnp.right_shift(word, sh), 255)

    def _compute(b):
        srcidx_b, dstidx_b, dist_b, sw_b, scidx_b, e1_b, e2_b = (
            srcidx[b], dstidx[b], dist[b], sw[b], scidx[b], e1[b], e2[b])

        @plsc.parallel_loop(0, CHUNK_E // 16, unroll=8)
        def _vec(v):
            sl = pl.ds(v * 16, 16)
            r = jnp.right_shift(v, 3)
            sl2 = pl.ds(jnp.left_shift(jnp.bitwise_and(v, 7), 4), 16)
            sraw = srcidx_b[r, sl2]
            scidx_b[sl] = sraw
            si = _species(sraw)
            sj = _species(dstidx_b[r, sl2])
            p = si * N_SPECIES + sj
            d = dist_b[r, sl2]
            swv = sw_b[r, sl2]
            mu = plsc.load_gather(tmu_v, [p])
            c6 = plsc.load_gather(tc6_v, [p])
            c8 = plsc.load_gather(tc8_v, [p])
            c10 = plsc.load_gather(tc10_v, [p])
            aq2 = plsc.load_gather(taq_v, [p])
            zn = mu * (d * d)
            ez = jnp.exp(zn)
            a1 = ez * zn
            a2 = a1 * zn
            a3 = a2 * zn
            a4 = a3 * zn
            a5 = a4 * zn
            f6 = 1.0 - ez + a1 - 0.5 * a2 + (1.0 / 6.0) * a3
            f8 = f6 - (1.0 / 24.0) * a4
            f10 = f8 + (1.0 / 120.0) * a5
            inv = 1.0 / d
            inv2 = inv * inv
            inv6 = inv2 * inv2 * inv2
            e1_b[sl] = swv * inv6 * (f6 * c6 + inv2 * (f8 * c8
                                                      + inv2 * (f10 * c10)))
            e2_b[sl] = aq2 * ez * inv * swv

    _fire_lin(0, 0)
    _fire_lin(1, 1)

    def _body(b, c):
        @pl.when(c < nchunks)
        def _():
            _wait_lin(b)
            _compute(b)
            _fire_scat(b)

            @pl.when(c >= 1)
            def _():
                _wait_scat((b + 2) % 3)

            @pl.when(c + 2 < nchunks)
            def _():
                _fire_lin((b + 2) % 3, c + 2)

    def _tri(i, carry):
        _body(0, 3 * i)
        _body(1, 3 * i + 1)
        _body(2, 3 * i + 2)
        return carry
    lax.fori_loop(0, (BASE_CHUNKS + 3) // 3, _tri, 0)

    for k in range(3):
        @pl.when((nchunks - 1) % 3 == k)
        def _(k=k):
            _wait_scat(k)

    plsc.subcore_barrier()
    seg = pl.ds(sid * SEG, SEG)
    oseg = pl.ds(cid * NPAD + sid * SEG, SEG)
    pltpu.sync_copy(acc1.at[seg], zbuf)
    pltpu.sync_copy(zbuf, edisp_o.at[oseg])
    pltpu.sync_copy(acc2.at[seg], zbuf)
    pltpu.sync_copy(zbuf, ex_o.at[oseg])


def _combine_body(e1_ref, e2_ref, edisp_ref, ex_ref, tot_ref):
    a = e1_ref[0] + e1_ref[1]
    b = e2_ref[0] + e2_ref[1]
    edisp_ref[...] = a
    ex_ref[...] = b
    tot_ref[...] = a + b


def kernel(species, edge_src, edge_dst, distances, switch):
    src1 = edge_src.astype(jnp.int32).reshape(N_ROWS, 128)
    dst1 = edge_dst.astype(jnp.int32).reshape(N_ROWS, 128)
    distances = distances.reshape(N_ROWS, 128)
    switch = switch.reshape(N_ROWS, 128)
    s4 = species.astype(jnp.int32).reshape(NSW, 4)
    specw = (s4[:, 0] | (s4[:, 1] << 8) | (s4[:, 2] << 16) | (s4[:, 3] << 24))

    mesh = plsc.VectorSubcoreMesh(core_axis_name="c", subcore_axis_name="s")
    f32 = jnp.float32
    sc = functools.partial(
        pl.kernel, mesh=mesh,
        compiler_params=pltpu.CompilerParams(needs_layout_passes=False),
        out_type=(jax.ShapeDtypeStruct((2 * NPAD,), f32),
                  jax.ShapeDtypeStruct((2 * NPAD,), f32)),
        scratch_types=[
            pltpu.VMEM((NSW,), jnp.int32),
            pltpu.VMEM((N_SPECIES * N_SPECIES,), f32),
            pltpu.VMEM((N_SPECIES * N_SPECIES,), f32),
            pltpu.VMEM((N_SPECIES * N_SPECIES,), f32),
            pltpu.VMEM((N_SPECIES * N_SPECIES,), f32),
            pltpu.VMEM((N_SPECIES * N_SPECIES,), f32),
        ] + [
            pltpu.VMEM((RPC, 128), jnp.int32),
            pltpu.VMEM((RPC, 128), jnp.int32),
            pltpu.VMEM((RPC, 128), f32),
            pltpu.VMEM((RPC, 128), f32),
            pltpu.VMEM((CHUNK_E,), jnp.int32),
            pltpu.VMEM((CHUNK_E,), f32),
            pltpu.VMEM((CHUNK_E,), f32),
        ] * 3 + [
            pltpu.VMEM((SEG,), f32),
            pltpu.VMEM_SHARED((NPAD,), f32),
            pltpu.VMEM_SHARED((NPAD,), f32),
            pltpu.SemaphoreType.DMA,
            pltpu.SemaphoreType.DMA,
            pltpu.SemaphoreType.DMA,
            pltpu.SemaphoreType.DMA,
            pltpu.SemaphoreType.DMA,
            pltpu.SemaphoreType.DMA,
        ])(_sc_body)

    edisp_p, ex_p = sc(
        specw, src1, dst1, distances, switch,
        jnp.asarray(_TBL_MUW), jnp.asarray(_TBL_C6), jnp.asarray(_TBL_C8),
        jnp.asarray(_TBL_C10), jnp.asarray(_TBL_AQ2))

    e1_3d = edisp_p.reshape(2, NPAD // 128, 128)
    e2_3d = ex_p.reshape(2, NPAD // 128, 128)
    edisp_pad, ex_pad, tot_pad = pl.pallas_call(
        _combine_body,
        out_shape=(jax.ShapeDtypeStruct((NPAD // 128, 128), f32),) * 3,
    )(e1_3d, e2_3d)
    edisp = edisp_pad.reshape(NPAD)[:N_NODES]
    ex = ex_pad.reshape(NPAD)[:N_NODES]
    tot = tot_pad.reshape(NPAD)[:N_NODES]
    return (edisp, ex, tot)


# speedup vs baseline: 1.0421x; 1.0203x over previous
import functools

import numpy as np
import jax
import jax.numpy as jnp
from jax import lax
from jax.experimental import pallas as pl
from jax.experimental.pallas import tpu as pltpu
from jax.experimental.pallas import tpu_sc as plsc

ANG = 0.52917721092
FSC = 0.0072973525693
N_NODES = 100000
N_EDGES = 6400000
N_SPECIES = 87

NW = 32
CHUNK_E = 2048
N_CHUNKS = N_EDGES // CHUNK_E
BASE_CHUNKS = N_CHUNKS // NW
EXTRA_CHUNKS = N_CHUNKS % NW
NPAD = 100096
SEG = NPAD // 16
NSW = N_NODES // 4
N_ROWS = N_EDGES // 128
RPC = CHUNK_E // 128


def _pair_tables():
    c6 = np.linspace(1.5, 500.0, N_SPECIES).astype(np.float32).astype(np.float64)
    al = np.linspace(1.0, 60.0, N_SPECIES).astype(np.float32).astype(np.float64)
    c6i, c6j = c6[:, None], c6[None, :]
    ai, aj = al[:, None], al[None, :]
    aij = 0.5 * (ai + aj)
    c6ij = 2 * ai * aj * c6i * c6j / (c6i * aj**2 + c6j * ai**2)
    Re = (aij * (128.0 / FSC ** (4.0 / 3.0))) ** (1.0 / 7.0)
    Re2 = Re * Re
    Re4 = Re2 * Re2
    muw = (0.483053463 - 0.0376191669 * Re + 0.00127066988 * Re2
           - 7.21940151e-07 * Re4) / (
        0.038421212 - 0.0316915319 * Re + 0.023741089 * Re2)
    c8ij = 5 * c6ij / muw
    c10ij = 245 * c6ij / (8 * muw**2)
    w = 4 * c6ij / (3 * aij**2)
    q2 = aij * muw * w
    ze = 0.5 * muw * Re2
    eze = np.exp(-ze)
    s6 = eze * (1.0 + ze + 0.5 * ze**2 + ze**3 / 6.0)
    f6e = 1.0 - s6
    muwRe = muw * Re
    df6e = muwRe * s6 - eze * (muwRe + 0.5 * Re * muwRe**2
                               + 1.0 / 8.0 * Re2 * muwRe**3)
    s8 = 1.0 / 24.0 * eze * ze**4
    f8e = f6e - s8
    df8e = df6e + muwRe * s8 - 1.0 / 48.0 * eze * Re2 * Re * muwRe**4
    s10 = 1.0 / 120.0 * eze * ze**5
    f10e = f8e - s10
    df10e = df8e + muwRe * s10 - 1.0 / 384.0 * eze * Re2 * Re2 * muwRe**5
    den = 2 * c6ij * Re2 * (6 * f6e - Re * df6e)
    A = (0.5 + c8ij * (8 * f8e - Re * df8e) / den
         + c10ij * (10 * f10e - Re * df10e) / (den * Re2))
    aq2 = A * q2
    MU = -0.5 * muw / ANG**2
    C6 = -0.5 * c6ij * ANG**6
    C8 = -0.5 * c8ij * ANG**8
    C10 = -0.5 * c10ij * ANG**10
    AQ = 0.5 * aq2 * ANG
    f32 = lambda x: np.asarray(x, np.float32)
    return (f32(MU).ravel(), f32(C6).ravel(), f32(C8).ravel(),
            f32(C10).ravel(), f32(AQ).ravel())


_TBL_MUW, _TBL_C6, _TBL_C8, _TBL_C10, _TBL_AQ2 = _pair_tables()


def _sc_body(specw_h, src_h, dst_h, dist_h, sw_h,
             tmu_h, tc6_h, tc8_h, tc10_h, taq_h,
             edisp_o, ex_o,
             specw_v, tmu_v, tc6_v, tc8_v, tc10_v, taq_v,
             srcidx_0, dstidx_0, dist_0, sw_0, scidx_0, e1_0, e2_0,
             srcidx_1, dstidx_1, dist_1, sw_1, scidx_1, e1_1, e2_1,
             srcidx_2, dstidx_2, dist_2, sw_2, scidx_2, e1_2, e2_2,
             zbuf, acc1, acc2,
             semL0, semL1, semL2, semS0, semS1, semS2):
    srcidx = (srcidx_0, srcidx_1, srcidx_2)
    dstidx = (dstidx_0, dstidx_1, dstidx_2)
    dist = (dist_0, dist_1, dist_2)
    sw = (sw_0, sw_1, sw_2)
    scidx = (scidx_0, scidx_1, scidx_2)
    e1 = (e1_0, e1_1, e1_2)
    e2 = (e2_0, e2_1, e2_2)
    semL = (semL0, semL1, semL2)
    semS = (semS0, semS1, semS2)
    cid = lax.axis_index("c")
    sid = lax.axis_index("s")
    wid = sid * 2 + cid

    pltpu.async_copy(specw_h, specw_v, semL0)
    pltpu.async_copy(tmu_h, tmu_v, semL0)
    pltpu.async_copy(tc6_h, tc6_v, semL0)
    pltpu.async_copy(tc8_h, tc8_v, semL0)
    pltpu.async_copy(tc10_h, tc10_v, semL0)
    pltpu.async_copy(taq_h, taq_v, semL0)

    def _zb(i, carry):
        zbuf[pl.ds(i * 16, 16)] = jnp.zeros((16,), jnp.float32)
        return carry
    lax.fori_loop(0, SEG // 16, _zb, 0)
    pltpu.sync_copy(zbuf, acc1.at[pl.ds(sid * SEG, SEG)])
    pltpu.sync_copy(zbuf, acc2.at[pl.ds(sid * SEG, SEG)])
    pltpu.make_async_copy(specw_h, specw_v, semL0).wait()
    pltpu.make_async_copy(tmu_h, tmu_v, semL0).wait()
    pltpu.make_async_copy(tc6_h, tc6_v, semL0).wait()
    pltpu.make_async_copy(tc8_h, tc8_v, semL0).wait()
    pltpu.make_async_copy(tc10_h, tc10_v, semL0).wait()
    pltpu.make_async_copy(taq_h, taq_v, semL0).wait()
    plsc.subcore_barrier()

    nchunks = BASE_CHUNKS + jnp.where(wid < EXTRA_CHUNKS, 1, 0)

    def _fire_lin(b, c):
        sl_e = pl.ds((wid + NW * c) * RPC, RPC)
        pltpu.async_copy(src_h.at[sl_e], srcidx[b], semL[b])
        pltpu.async_copy(dst_h.at[sl_e], dstidx[b], semL[b])
        pltpu.async_copy(dist_h.at[sl_e], dist[b], semL[b])
        pltpu.async_copy(sw_h.at[sl_e], sw[b], semL[b])

    def _wait_lin(b):
        sl_e = pl.ds(0, RPC)
        pltpu.make_async_copy(src_h.at[sl_e], srcidx[b], semL[b]).wait()
        pltpu.make_async_copy(dst_h.at[sl_e], dstidx[b], semL[b]).wait()
        pltpu.make_async_copy(dist_h.at[sl_e], dist[b], semL[b]).wait()
        pltpu.make_async_copy(sw_h.at[sl_e], sw[b], semL[b]).wait()

    def _fire_scat(b):
        pltpu.async_copy(e1[b], acc1.at[scidx[b]], semS[b], add=True)
        pltpu.async_copy(e2[b], acc2.at[scidx[b]], semS[b], add=True)

    def _wait_scat(b):
        pltpu.make_async_copy(e1[b], acc1.at[scidx[b]], semS[b]).wait()
        pltpu.make_async_copy(e2[b], acc2.at[scidx[b]], semS[b]).wait()

    def _species(idx_vec):
        word = plsc.load_gather(specw_v, [jnp.right_shift(idx_vec, 2)])
        sh = jnp.left_shift(jnp.bitwise_and(idx_vec, 3), 3)
        return jnp.bitwise_and(jnp.right_shift(word, sh), 255)

    def _compute(b):
        srcidx_b, dstidx_b, dist_b, sw_b, scidx_b, e1_b, e2_b = (
            srcidx[b], dstidx[b], dist[b], sw[b], scidx[b], e1[b], e2[b])

        @plsc.parallel_loop(0, CHUNK_E // 16, unroll=8)
        def _vec(v):
            sl = pl.ds(v * 16, 16)
            r = jnp.right_shift(v, 3)
            sl2 = pl.ds(jnp.left_shift(jnp.bitwise_and(v, 7), 4), 16)
            sraw = srcidx_b[r, sl2]
            scidx_b[sl] = sraw
            si = _species(sraw)
            sj = _species(dstidx_b[r, sl2])
            p = si * N_SPECIES + sj
            d = dist_b[r, sl2]
            swv = sw_b[r, sl2]
            mu = plsc.load_gather(tmu_v, [p])
            c6 = plsc.load_gather(tc6_v, [p])
            c8 = plsc.load_gather(tc8_v, [p])
            c10 = plsc.load_gather(tc10_v, [p])
            aq2 = plsc.load_gather(taq_v, [p])
            zn = mu * (d * d)
            ez = jnp.exp(zn)
            a1 = ez * zn
            a2 = a1 * zn
            a3 = a2 * zn
            a4 = a3 * zn
            a5 = a4 * zn
            f6 = 1.0 - ez + a1 - 0.5 * a2 + (1.0 / 6.0) * a3
            f8 = f6 - (1.0 / 24.0) * a4
            f10 = f8 + (1.0 / 120.0) * a5
            inv = 1.0 / d
            inv2 = inv * inv
            inv6 = inv2 * inv2 * inv2
            e1_b[sl] = swv * inv6 * (f6 * c6 + inv2 * (f8 * c8
                                                      + inv2 * (f10 * c10)))
            e2_b[sl] = aq2 * ez * inv * swv

    _fire_lin(0, 0)
    _fire_lin(1, 1)

    def _body(b, c):
        @pl.when(c < nchunks)
        def _():
            _wait_lin(b)
            _compute(b)
            _fire_scat(b)

            @pl.when(c >= 1)
            def _():
                _wait_scat((b + 2) % 3)

            @pl.when(c + 2 < nchunks)
            def _():
                _fire_lin((b + 2) % 3, c + 2)

    def _tri(i, carry):
        _body(0, 3 * i)
        _body(1, 3 * i + 1)
        _body(2, 3 * i + 2)
        return carry
    lax.fori_loop(0, (BASE_CHUNKS + 3) // 3, _tri, 0)

    for k in range(3):
        @pl.when((nchunks - 1) % 3 == k)
        def _(k=k):
            _wait_scat(k)

    plsc.subcore_barrier()
    seg = pl.ds(sid * SEG, SEG)
    oseg = pl.ds(cid * NPAD + sid * SEG, SEG)
    pltpu.sync_copy(acc1.at[seg], zbuf)
    pltpu.sync_copy(zbuf, edisp_o.at[oseg])
    pltpu.sync_copy(acc2.at[seg], zbuf)
    pltpu.sync_copy(zbuf, ex_o.at[oseg])


def _combine_body(e1_ref, e2_ref, edisp_ref, ex_ref, tot_ref):
    a = e1_ref[0] + e1_ref[1]
    b = e2_ref[0] + e2_ref[1]
    edisp_ref[...] = a
    ex_ref[...] = b
    tot_ref[...] = a + b


def kernel(species, edge_src, edge_dst, distances, switch):
    src1 = edge_src.astype(jnp.int32).reshape(N_ROWS, 128)
    dst1 = edge_dst.astype(jnp.int32).reshape(N_ROWS, 128)
    distances = distances.reshape(N_ROWS, 128)
    switch = switch.reshape(N_ROWS, 128)
    s4 = species.astype(jnp.int32).reshape(NSW, 4)
    specw = (s4[:, 0] | (s4[:, 1] << 8) | (s4[:, 2] << 16) | (s4[:, 3] << 24))

    mesh = plsc.VectorSubcoreMesh(core_axis_name="c", subcore_axis_name="s")
    f32 = jnp.float32
    sc = functools.partial(
        pl.kernel, mesh=mesh,
        compiler_params=pltpu.CompilerParams(needs_layout_passes=False),
        out_type=(jax.ShapeDtypeStruct((2 * NPAD,), f32),
                  jax.ShapeDtypeStruct((2 * NPAD,), f32)),
        scratch_types=[
            pltpu.VMEM((NSW,), jnp.int32),
            pltpu.VMEM((N_SPECIES * N_SPECIES,), f32),
            pltpu.VMEM((N_SPECIES * N_SPECIES,), f32),
            pltpu.VMEM((N_SPECIES * N_SPECIES,), f32),
            pltpu.VMEM((N_SPECIES * N_SPECIES,), f32),
            pltpu.VMEM((N_SPECIES * N_SPECIES,), f32),
        ] + [
            pltpu.VMEM((RPC, 128), jnp.int32),
            pltpu.VMEM((RPC, 128), jnp.int32),
            pltpu.VMEM((RPC, 128), f32),
            pltpu.VMEM((RPC, 128), f32),
            pltpu.VMEM((CHUNK_E,), jnp.int32),
            pltpu.VMEM((CHUNK_E,), f32),
            pltpu.VMEM((CHUNK_E,), f32),
        ] * 3 + [
            pltpu.VMEM((SEG,), f32),
            pltpu.VMEM_SHARED((NPAD,), f32),
            pltpu.VMEM_SHARED((NPAD,), f32),
            pltpu.SemaphoreType.DMA,
            pltpu.SemaphoreType.DMA,
            pltpu.SemaphoreType.DMA,
            pltpu.SemaphoreType.DMA,
            pltpu.SemaphoreType.DMA,
            pltpu.SemaphoreType.DMA,
        ])(_sc_body)

    edisp_p, ex_p = sc(
        specw, src1, dst1, distances, switch,
        jnp.asarray(_TBL_MUW), jnp.asarray(_TBL_C6), jnp.asarray(_TBL_C8),
        jnp.asarray(_TBL_C10), jnp.asarray(_TBL_AQ2))

    e1_3d = edisp_p.reshape(2, NPAD // 128, 128)
    e2_3d = ex_p.reshape(2, NPAD // 128, 128)
    edisp_pad, ex_pad, tot_pad = pl.pallas_call(
        _combine_body,
        out_shape=(jax.ShapeDtypeStruct((NPAD // 128, 128), f32),) * 3,
    )(e1_3d, e2_3d)
    edisp = edisp_pad.reshape(NPAD)[:N_NODES]
    ex = ex_pad.reshape(NPAD)[:N_NODES]
    tot = tot_pad.reshape(NPAD)[:N_NODES]
    return (edisp, ex, tot)
